# baseline (speedup 1.0000x reference)
import functools

import jax
import jax.numpy as jnp
from jax.experimental import pallas as pl
from jax.experimental.pallas import tpu as pltpu


def _round_up(x, m):
    return ((x + m - 1) // m) * m


def _pad(a, target_shape):
    widths = [(0, t - s) for s, t in zip(a.shape, target_shape)]
    if all(w == (0, 0) for w in widths):
        return a
    return jnp.pad(a, widths)


def _fused_kernel(x_ref, w1_ref, b1_ref, w2_ref, b2_ref, wp_ref, bp_ref,
                  o_ref, acc_ref, w1_s, w2_s, wp_s,
                  *, inv_t, cj_in, cj_out):
    t = pl.program_id(1)

    @pl.when(t == 0)
    def _():
        acc_ref[...] = jnp.zeros_like(acc_ref)

    acc_ref[...] += jnp.sum(x_ref[...].astype(jnp.float32), axis=1)

    w1_s[:, pl.ds(t * cj_in, cj_in)] = w1_ref[...].astype(jnp.bfloat16)
    w2_s[:, pl.ds(t * cj_in, cj_in)] = w2_ref[...].astype(jnp.bfloat16)
    wp_s[:, pl.ds(t * cj_out, cj_out)] = wp_ref[...].astype(jnp.bfloat16)

    @pl.when(t == pl.num_programs(1) - 1)
    def _():
        pooled = (acc_ref[...] * inv_t).astype(jnp.bfloat16)
        h = jnp.dot(pooled, w1_s[...], preferred_element_type=jnp.float32)
        h = jnp.maximum(h + b1_ref[...], 0.0).astype(jnp.bfloat16)
        h = jnp.dot(h, w2_s[...], preferred_element_type=jnp.float32)
        h = (h + b2_ref[...]).astype(jnp.bfloat16)
        out = jnp.dot(h, wp_s[...], preferred_element_type=jnp.float32)
        o_ref[...] = (out + bp_ref[...]).astype(o_ref.dtype)


def _resident(shape, index_map):
    return pl.BlockSpec(shape, index_map, pipeline_mode=pl.Buffered(1))


def kernel(x, w1, b1, w2, b2, w_proj, b_proj):
    B, T, D_in = x.shape
    D_out = w_proj.shape[1]
    itemsize = jnp.dtype(x.dtype).itemsize

    if B >= 16:
        TB = _round_up((B + 1) // 2, 8)
    else:
        TB = _round_up(max(B, 1), 8)
    B_pad = _round_up(B, TB)

    TT = max(8, (9 * 1024 * 1024) // (TB * _round_up(D_in, 128) * itemsize)
             // 8 * 8)
    TT = min(TT, _round_up(T, 8))
    T_pad = _round_up(T, TT)
    NT = T_pad // TT

    D_in_p = _round_up(D_in, 128)
    D_out_p = _round_up(D_out, 128)
    cj_in = _round_up(-(-D_in_p // NT), 128)
    cj_out = _round_up(-(-D_out_p // NT), 128)
    D_in_c = NT * cj_in
    D_out_c = NT * cj_out

    x_p = _pad(x, (B_pad, T_pad, D_in_p))
    w1p = _pad(w1, (D_in_p, D_in_c))
    b1p = _pad(b1.reshape(1, -1), (1, D_in_p))
    w2p = _pad(w2, (D_in_p, D_in_c))
    b2p = _pad(b2.reshape(1, -1), (1, D_in_p))
    wpp = _pad(w_proj, (D_in_p, D_out_c))
    bpp = _pad(b_proj.reshape(1, -1), (1, D_out_c))

    grid = (B_pad // TB, NT)
    weight_bytes = (2 * D_in_p * D_in_c + D_in_p * D_out_c) * 4
    bytes_accessed = (x_p.size * itemsize + weight_bytes
                      + B_pad * D_out_c * itemsize)
    flops = (B_pad * T_pad * D_in_p + 4 * B_pad * D_in_p * D_in_p
             + 2 * B_pad * D_in_p * D_out_p)

    out = pl.pallas_call(
        functools.partial(_fused_kernel, inv_t=1.0 / T,
                          cj_in=cj_in, cj_out=cj_out),
        out_shape=jax.ShapeDtypeStruct((B_pad, D_out_c), x.dtype),
        grid=grid,
        in_specs=[
            pl.BlockSpec((TB, TT, D_in_p), lambda b, t: (b, t, 0)),
            pl.BlockSpec((D_in_p, cj_in), lambda b, t: (0, t)),
            _resident((1, D_in_p), lambda b, t: (0, 0)),
            pl.BlockSpec((D_in_p, cj_in), lambda b, t: (0, t)),
            _resident((1, D_in_p), lambda b, t: (0, 0)),
            pl.BlockSpec((D_in_p, cj_out), lambda b, t: (0, t)),
            _resident((1, D_out_c), lambda b, t: (0, 0)),
        ],
        out_specs=pl.BlockSpec((TB, D_out_c), lambda b, t: (b, 0)),
        scratch_shapes=[
            pltpu.VMEM((TB, D_in_p), jnp.float32),
            pltpu.VMEM((D_in_p, D_in_c), jnp.bfloat16),
            pltpu.VMEM((D_in_p, D_in_c), jnp.bfloat16),
            pltpu.VMEM((D_in_p, D_out_c), jnp.bfloat16),
        ],
        compiler_params=pltpu.CompilerParams(
            dimension_semantics=("parallel", "arbitrary"),
            vmem_limit_bytes=56 * 1024 * 1024),
        cost_estimate=pl.CostEstimate(
            flops=int(flops), transcendentals=0,
            bytes_accessed=int(bytes_accessed)),
    )(x_p, w1p, b1p, w2p, b2p, wpp, bpp)

    return out[:B, None, :D_out]


# speedup vs baseline: 1.1761x; 1.1761x over previous
import functools

import jax
import jax.numpy as jnp
from jax.experimental import pallas as pl
from jax.experimental.pallas import tpu as pltpu


def _round_up(x, m):
    return ((x + m - 1) // m) * m


def _pad(a, target_shape):
    widths = [(0, t - s) for s, t in zip(a.shape, target_shape)]
    if all(w == (0, 0) for w in widths):
        return a
    return jnp.pad(a, widths)


def _fused_kernel(x_ref, w1_ref, b1_ref, w2_ref, b2_ref, wp_ref, bp_ref,
                  o_ref, acc_ref, *, inv_t):
    t = pl.program_id(1)

    @pl.when(t == 0)
    def _():
        acc_ref[...] = jnp.zeros_like(acc_ref)

    acc_ref[...] += jnp.sum(x_ref[...].astype(jnp.float32), axis=1)

    @pl.when(t == pl.num_programs(1) - 1)
    def _():
        pooled = (acc_ref[...] * inv_t).astype(w1_ref.dtype)
        h = jnp.dot(pooled, w1_ref[...], preferred_element_type=jnp.float32)
        h = jnp.maximum(h + b1_ref[...], 0.0).astype(w2_ref.dtype)
        h = jnp.dot(h, w2_ref[...], preferred_element_type=jnp.float32)
        h = (h + b2_ref[...]).astype(wp_ref.dtype)
        out = jnp.dot(h, wp_ref[...], preferred_element_type=jnp.float32)
        o_ref[...] = (out + bp_ref[...]).astype(o_ref.dtype)


def _resident(shape, index_map):
    return pl.BlockSpec(shape, index_map, pipeline_mode=pl.Buffered(1))


def kernel(x, w1, b1, w2, b2, w_proj, b_proj):
    B, T, D_in = x.shape
    D_out = w_proj.shape[1]
    D_in_p = _round_up(D_in, 128)
    D_out_p = _round_up(D_out, 128)
    itemsize = jnp.dtype(x.dtype).itemsize

    if B >= 16:
        TB = _round_up((B + 1) // 2, 8)
    else:
        TB = _round_up(max(B, 1), 8)
    B_pad = _round_up(B, TB)

    TT = max(8, (9 * 1024 * 1024) // (TB * D_in_p * itemsize) // 8 * 8)
    TT = min(TT, _round_up(T, 8))
    T_pad = _round_up(T, TT)

    x_p = _pad(x, (B_pad, T_pad, D_in_p))
    w1p = _pad(w1, (D_in_p, D_in_p))
    b1p = _pad(b1.reshape(1, -1), (1, D_in_p))
    w2p = _pad(w2, (D_in_p, D_in_p))
    b2p = _pad(b2.reshape(1, -1), (1, D_in_p))
    wpp = _pad(w_proj, (D_in_p, D_out_p))
    bpp = _pad(b_proj.reshape(1, -1), (1, D_out_p))

    grid = (B_pad // TB, T_pad // TT)
    weight_bytes = (2 * D_in_p * D_in_p + D_in_p * D_out_p) * 4
    bytes_accessed = (x_p.size * itemsize + weight_bytes
                      + B_pad * D_out_p * itemsize)
    flops = (B_pad * T_pad * D_in_p + 4 * B_pad * D_in_p * D_in_p
             + 2 * B_pad * D_in_p * D_out_p)

    out = pl.pallas_call(
        functools.partial(_fused_kernel, inv_t=1.0 / T),
        out_shape=jax.ShapeDtypeStruct((B_pad, D_out_p), x.dtype),
        grid=grid,
        in_specs=[
            pl.BlockSpec((TB, TT, D_in_p), lambda b, t: (b, t, 0)),
            _resident((D_in_p, D_in_p), lambda b, t: (0, 0)),
            _resident((1, D_in_p), lambda b, t: (0, 0)),
            _resident((D_in_p, D_in_p), lambda b, t: (0, 0)),
            _resident((1, D_in_p), lambda b, t: (0, 0)),
            _resident((D_in_p, D_out_p), lambda b, t: (0, 0)),
            _resident((1, D_out_p), lambda b, t: (0, 0)),
        ],
        out_specs=pl.BlockSpec((TB, D_out_p), lambda b, t: (b, 0)),
        scratch_shapes=[pltpu.VMEM((TB, D_in_p), jnp.float32)],
        compiler_params=pltpu.CompilerParams(
            dimension_semantics=("parallel", "arbitrary"),
            vmem_limit_bytes=56 * 1024 * 1024),
        cost_estimate=pl.CostEstimate(
            flops=int(flops), transcendentals=0,
            bytes_accessed=int(bytes_accessed)),
    )(x_p, w1p, b1p, w2p, b2p, wpp, bpp)

    return out[:B, None, :D_out]
